# Initial kernel scaffold; baseline (speedup 1.0000x reference)
import jax
import jax.numpy as jnp
from jax.experimental import pallas as pl


def kernel(inputs, emb, W0, b0, W1, b1, W2, b2, Wf, bf):
    raise NotImplementedError("write your pallas kernel here")


# speedup vs baseline: 7.9029x; 7.9029x over previous
import functools

import jax
import jax.numpy as jnp
from jax import lax
from jax.experimental import pallas as pl
from jax.experimental.pallas import tpu as pltpu
from jax.experimental.pallas import tpu_sc as plsc

_B = 16384
_M = 26
_V = 100000
_D = 16
_H = 16

_R = _B * _M
_NC = 2
_NS = 16
_NW = _NC * _NS
_PER_W = _R // _NW
_IDX_ROW = 128
_SUB = 13
_CHUNK = _SUB * _IDX_ROW
_NCHUNK = _PER_W // _CHUNK
_IDX_ROWS_W = _PER_W // _IDX_ROW


def _sc_gather(idx2, table):
    mesh = plsc.VectorSubcoreMesh(core_axis_name="c", subcore_axis_name="s")

    @functools.partial(
        pl.kernel,
        mesh=mesh,
        compiler_params=pltpu.CompilerParams(use_tc_tiling_on_sc=False),
        out_type=jax.ShapeDtypeStruct((_R, _D), jnp.float32),
        scratch_types=[
            pltpu.VMEM((_IDX_ROWS_W, _IDX_ROW), jnp.int32),
            pltpu.VMEM((_CHUNK, _D), jnp.float32),
            pltpu.SemaphoreType.DMA,
        ],
    )
    def gather_kernel(idx_hbm, table_hbm, out_hbm, idx_v, rows_v, sem):
        wid = lax.axis_index("s") * _NC + lax.axis_index("c")
        pltpu.sync_copy(idx_hbm.at[pl.ds(wid * _IDX_ROWS_W, _IDX_ROWS_W)], idx_v)
        for i in range(_NCHUNK):
            handles = []
            for j in range(_SUB):
                handles.append(
                    pltpu.async_copy(
                        table_hbm.at[idx_v.at[i * _SUB + j]],
                        rows_v.at[pl.ds(j * _IDX_ROW, _IDX_ROW)],
                        sem,
                    )
                )
            for h in handles:
                h.wait()
            out0 = wid * _PER_W + i * _CHUNK
            pltpu.sync_copy(rows_v, out_hbm.at[pl.ds(out0, _CHUNK)])

    return gather_kernel(idx2, table)


_BB = 2048


def _mlp_block(x_ref, w0_ref, b0_ref, w1_ref, b1_ref, w2_ref, b2_ref,
               wf_ref, bf_ref, o_ref):
    x = x_ref[...]
    h = jnp.maximum(
        jnp.dot(x, w0_ref[...], preferred_element_type=jnp.float32) + b0_ref[...], 0.0)
    h = jnp.maximum(
        jnp.dot(h, w1_ref[...], preferred_element_type=jnp.float32) + b1_ref[...], 0.0) + h
    h = jnp.maximum(
        jnp.dot(h, w2_ref[...], preferred_element_type=jnp.float32) + b2_ref[...], 0.0) + h
    y = jnp.dot(h, wf_ref[...], preferred_element_type=jnp.float32) + bf_ref[...]
    o_ref[...] = -y[:, 0]


def _mlp(x, W0, b0, W1, b1, W2, b2, Wf, bf):
    full = lambda a: pl.BlockSpec(a.shape, lambda i: (0,) * a.ndim)
    return pl.pallas_call(
        _mlp_block,
        grid=(_B // _BB,),
        in_specs=[
            pl.BlockSpec((_BB, _M * _D), lambda i: (i, 0)),
            full(W0), full(b0), full(W1), full(b1), full(W2), full(b2),
            full(Wf), full(bf),
        ],
        out_specs=pl.BlockSpec((_BB,), lambda i: (i,)),
        out_shape=jax.ShapeDtypeStruct((_B,), jnp.float32),
    )(x, W0, b0, W1, b1, W2, b2, Wf, bf)


def kernel(inputs, emb, W0, b0, W1, b1, W2, b2, Wf, bf):
    flat_idx = (inputs + (jnp.arange(_M, dtype=jnp.int32) * _V)[None, :]).reshape(-1)
    idx2 = flat_idx.reshape(_R // _IDX_ROW, _IDX_ROW)
    table = emb.reshape(_M * _V, _D)
    X = _sc_gather(idx2, table)
    x = X.reshape(_B, _M * _D)
    return _mlp(x, W0, b0.reshape(1, _H), W1, b1.reshape(1, _H),
                W2, b2.reshape(1, _H), Wf, bf.reshape(1, 1))


# speedup vs baseline: 22.9314x; 2.9017x over previous
import functools

import jax
import jax.numpy as jnp
from jax import lax
from jax.experimental import pallas as pl
from jax.experimental.pallas import tpu as pltpu
from jax.experimental.pallas import tpu_sc as plsc

_B = 16384
_M = 26
_V = 100000
_D = 16
_H = 16

_NC = 2
_NS = 16
_NW = _NC * _NS
_BW = _B // _NW
_IDX_ROW = 128
_SUB = _BW // _IDX_ROW

_TSB = 1024
_TBT = 8192
_TGRID = -(-_V // _TBT)
_SBM = _TGRID * 8
_VPAD = _SBM * _TSB


def _tr_block(e_ref, o_ref):
    for s in range(8):
        ec = e_ref[0, :, s * _TSB:(s + 1) * _TSB]
        f = jnp.concatenate(
            [ec[:, a * 128:(a + 1) * 128] for a in range(8)], axis=0)
        o_ref[0, s, :, :] = f.T


def _format_table(embT):
    return pl.pallas_call(
        _tr_block,
        grid=(_M, _TGRID),
        in_specs=[pl.BlockSpec((1, _D, _TBT), lambda m, c: (m, 0, c))],
        out_specs=pl.BlockSpec((1, 8, 128, 128), lambda m, c: (m, c, 0, 0)),
        out_shape=jax.ShapeDtypeStruct((_M, _SBM, 128, 128), jnp.float32),
    )(embT)


def _sc_gather(idxT, table):
    mesh = plsc.VectorSubcoreMesh(core_axis_name="c", subcore_axis_name="s")

    @functools.partial(
        pl.kernel,
        mesh=mesh,
        compiler_params=pltpu.CompilerParams(use_tc_tiling_on_sc=False),
        out_type=jax.ShapeDtypeStruct((_B, _M * _D), jnp.float32),
        scratch_types=[
            pltpu.VMEM((_M, _BW), jnp.int32),
            pltpu.VMEM((2, _BW, _D), jnp.float32),
            pltpu.SemaphoreType.DMA,
            pltpu.SemaphoreType.DMA,
        ],
    )
    def gather_kernel(idx_hbm, table_hbm, out_hbm, idx_v, rows_v, sem0, sem1):
        wid = lax.axis_index("s") * _NC + lax.axis_index("c")
        b0 = wid * _BW
        pltpu.sync_copy(idx_hbm.at[:, pl.ds(b0, _BW)], idx_v)
        sems = (sem0, sem1)
        stores = [None, None]
        for m in range(_M):
            s = m % 2
            handles = []
            for j in range(_SUB):
                handles.append(
                    pltpu.async_copy(
                        table_hbm.at[pl.ds(m * _VPAD, _V)].at[
                            idx_v.at[m, pl.ds(j * _IDX_ROW, _IDX_ROW)]],
                        rows_v.at[s].at[pl.ds(j * _IDX_ROW, _IDX_ROW)],
                        sems[s],
                    )
                )
            if stores[s] is not None:
                stores[s].wait()
            for h in handles:
                h.wait()
            stores[s] = pltpu.async_copy(
                rows_v.at[s],
                out_hbm.at[pl.ds(b0, _BW), pl.ds(m * _D, _D)],
                sems[s],
            )
        for st in stores:
            if st is not None:
                st.wait()

    return gather_kernel(idxT, table)


_BB = 2048


def _mlp_block(x_ref, w0_ref, b0_ref, w1_ref, b1_ref, w2_ref, b2_ref,
               wf_ref, bf_ref, o_ref):
    x = x_ref[...]
    h = jnp.maximum(
        jnp.dot(x, w0_ref[...], preferred_element_type=jnp.float32) + b0_ref[...], 0.0)
    h = jnp.maximum(
        jnp.dot(h, w1_ref[...], preferred_element_type=jnp.float32) + b1_ref[...], 0.0) + h
    h = jnp.maximum(
        jnp.dot(h, w2_ref[...], preferred_element_type=jnp.float32) + b2_ref[...], 0.0) + h
    y = jnp.dot(h, wf_ref[...], preferred_element_type=jnp.float32) + bf_ref[...]
    o_ref[...] = -y[:, 0]


def _mlp(x, W0, b0, W1, b1, W2, b2, Wf, bf):
    full = lambda a: pl.BlockSpec(a.shape, lambda i: (0,) * a.ndim)
    return pl.pallas_call(
        _mlp_block,
        grid=(_B // _BB,),
        in_specs=[
            pl.BlockSpec((_BB, _M * _D), lambda i: (i, 0)),
            full(W0), full(b0), full(W1), full(b1), full(W2), full(b2),
            full(Wf), full(bf),
        ],
        out_specs=pl.BlockSpec((_BB,), lambda i: (i,)),
        out_shape=jax.ShapeDtypeStruct((_B,), jnp.float32),
    )(x, W0, b0, W1, b1, W2, b2, Wf, bf)


def kernel(inputs, emb, W0, b0, W1, b1, W2, b2, Wf, bf):
    wide = _format_table(jnp.transpose(emb, (0, 2, 1)))
    t = inputs.T
    rows = (t & -1024) + ((t & 127) << 3) + ((t >> 7) & 7)
    x = _sc_gather(rows, wide.reshape(_M * _VPAD, _D))
    return _mlp(x, W0, b0.reshape(1, _H), W1, b1.reshape(1, _H),
                W2, b2.reshape(1, _H), Wf, bf.reshape(1, 1))


# speedup vs baseline: 31.8866x; 1.3905x over previous
import functools

import jax
import jax.numpy as jnp
from jax import lax
from jax.experimental import pallas as pl
from jax.experimental.pallas import tpu as pltpu
from jax.experimental.pallas import tpu_sc as plsc

_B = 16384
_M = 26
_V = 100000
_D = 16
_H = 16

_NC = 2
_NS = 16
_NW = _NC * _NS
_BW = _B // _NW
_IDX_ROW = 128
_SUB = _BW // _IDX_ROW

_TSB = 1024
_TBT = 8192
_TGRID = -(-_V // _TBT)
_SBM = _TGRID * 8
_VPAD = _SBM * _TSB

_MA = 14
_MB = _M - _MA


def _tr_block(e0_ref, e1_ref, o0_ref, o1_ref):
    for e_ref, o_ref in ((e0_ref, o0_ref), (e1_ref, o1_ref)):
        for s in range(8):
            ec = e_ref[0, :, s * _TSB:(s + 1) * _TSB]
            f = jnp.concatenate(
                [ec[:, a * 128:(a + 1) * 128] for a in range(8)], axis=0)
            o_ref[0, s, :, :] = f.T


def _format_table(embT, a0, na):
    ng = na // 2
    sd = jax.ShapeDtypeStruct((ng, _SBM, 128, 128), jnp.float32)
    return pl.pallas_call(
        _tr_block,
        grid=(ng, _TGRID),
        in_specs=[
            pl.BlockSpec((1, _D, _TBT), lambda g, c: (a0 + 2 * g, 0, c)),
            pl.BlockSpec((1, _D, _TBT), lambda g, c: (a0 + 2 * g + 1, 0, c)),
        ],
        out_specs=[
            pl.BlockSpec((1, 8, 128, 128), lambda g, c: (g, c, 0, 0)),
            pl.BlockSpec((1, 8, 128, 128), lambda g, c: (g, c, 0, 0)),
        ],
        out_shape=(sd, sd),
    )(embT, embT)


def _sc_gather(idxT, tev, tod, na):
    mesh = plsc.VectorSubcoreMesh(core_axis_name="c", subcore_axis_name="s")

    @functools.partial(
        pl.kernel,
        mesh=mesh,
        compiler_params=pltpu.CompilerParams(use_tc_tiling_on_sc=False),
        out_type=jax.ShapeDtypeStruct((_B, na * _D), jnp.float32),
        scratch_types=[
            pltpu.VMEM((na, _BW), jnp.int32),
            pltpu.VMEM((2, _BW, _D), jnp.float32),
            pltpu.SemaphoreType.DMA,
            pltpu.SemaphoreType.DMA,
        ],
    )
    def gather_kernel(idx_hbm, tev_hbm, tod_hbm, out_hbm, idx_v, rows_v,
                      sem0, sem1):
        wid = lax.axis_index("s") * _NC + lax.axis_index("c")
        b0 = wid * _BW
        pltpu.sync_copy(idx_hbm.at[:, pl.ds(b0, _BW)], idx_v)
        sems = (sem0, sem1)

        def issue(m):
            tab = tev_hbm if m % 2 == 0 else tod_hbm
            base = (m // 2) * _VPAD
            s = m % 2
            return [
                pltpu.async_copy(
                    tab.at[pl.ds(base, _VPAD)].at[
                        idx_v.at[m, pl.ds(j * _IDX_ROW, _IDX_ROW)]],
                    rows_v.at[s].at[pl.ds(j * _IDX_ROW, _IDX_ROW)],
                    sems[s],
                )
                for j in range(_SUB)
            ]

        gathers = issue(0)
        stores = [None, None]
        for m in range(na):
            s = m % 2
            nxt = None
            if m + 1 < na:
                if stores[1 - s] is not None:
                    stores[1 - s].wait()
                    stores[1 - s] = None
                nxt = issue(m + 1)
            for h in gathers:
                h.wait()
            gathers = nxt
            if stores[s] is not None:
                stores[s].wait()
            stores[s] = pltpu.async_copy(
                rows_v.at[s],
                out_hbm.at[pl.ds(b0, _BW), pl.ds(m * _D, _D)],
                sems[s],
            )
        for st in stores:
            if st is not None:
                st.wait()

    return gather_kernel(idxT, tev, tod)


_BB = 2048


def _mlp_block(xa_ref, xb_ref, w0a_ref, w0b_ref, b0_ref, w1_ref, b1_ref,
               w2_ref, b2_ref, wf_ref, bf_ref, o_ref):
    h = (jnp.dot(xa_ref[...], w0a_ref[...], preferred_element_type=jnp.float32)
         + jnp.dot(xb_ref[...], w0b_ref[...], preferred_element_type=jnp.float32)
         + b0_ref[...])
    h = jnp.maximum(h, 0.0)
    h = jnp.maximum(
        jnp.dot(h, w1_ref[...], preferred_element_type=jnp.float32) + b1_ref[...], 0.0) + h
    h = jnp.maximum(
        jnp.dot(h, w2_ref[...], preferred_element_type=jnp.float32) + b2_ref[...], 0.0) + h
    y = jnp.dot(h, wf_ref[...], preferred_element_type=jnp.float32) + bf_ref[...]
    o_ref[...] = -y[:, 0]


def _mlp(xa, xb, W0a, W0b, b0, W1, b1, W2, b2, Wf, bf):
    full = lambda a: pl.BlockSpec(a.shape, lambda i: (0,) * a.ndim)
    return pl.pallas_call(
        _mlp_block,
        grid=(_B // _BB,),
        in_specs=[
            pl.BlockSpec((_BB, _MA * _D), lambda i: (i, 0)),
            pl.BlockSpec((_BB, _MB * _D), lambda i: (i, 0)),
            full(W0a), full(W0b), full(b0), full(W1), full(b1),
            full(W2), full(b2), full(Wf), full(bf),
        ],
        out_specs=pl.BlockSpec((_BB,), lambda i: (i,)),
        out_shape=jax.ShapeDtypeStruct((_B,), jnp.float32),
    )(xa, xb, W0a, W0b, b0, W1, b1, W2, b2, Wf, bf)


def kernel(inputs, emb, W0, b0, W1, b1, W2, b2, Wf, bf):
    embT = jnp.transpose(emb, (0, 2, 1))
    t = inputs.T
    rows = (t & -1024) + ((t & 127) << 3) + ((t >> 7) & 7)
    ta_ev, ta_od = _format_table(embT, 0, _MA)
    tb_ev, tb_od = _format_table(embT, _MA, _MB)
    xa = _sc_gather(rows[:_MA], ta_ev.reshape(-1, _D), ta_od.reshape(-1, _D), _MA)
    xb = _sc_gather(rows[_MA:], tb_ev.reshape(-1, _D), tb_od.reshape(-1, _D), _MB)
    return _mlp(xa, xb, W0[:_MA * _D], W0[_MA * _D:], b0.reshape(1, _H),
                W1, b1.reshape(1, _H), W2, b2.reshape(1, _H), Wf,
                bf.reshape(1, 1))


# speedup vs baseline: 34.0523x; 1.0679x over previous
import functools

import jax
import jax.numpy as jnp
from jax import lax
from jax.experimental import pallas as pl
from jax.experimental.pallas import tpu as pltpu
from jax.experimental.pallas import tpu_sc as plsc

_B = 16384
_M = 26
_V = 100000
_D = 16
_H = 16

_NC = 2
_NS = 16
_NW = _NC * _NS
_BW = _B // _NW
_IDX_ROW = 128
_SUB = _BW // _IDX_ROW

_TSB = 1024
_TBT = 8192
_TGRID = -(-_V // _TBT)
_SBM = _TGRID * 8
_VPAD = _SBM * _TSB

_MA = 16
_MB = _M - _MA
_PB = 8


def _tr_block(e0_ref, e1_ref, o0_ref, o1_ref):
    for e_ref, o_ref in ((e0_ref, o0_ref), (e1_ref, o1_ref)):
        for s in range(8):
            ec = e_ref[0, :, s * _TSB:(s + 1) * _TSB]
            f = jnp.concatenate(
                [ec[:, a * 128:(a + 1) * 128] for a in range(8)], axis=0)
            o_ref[0, s, :, :] = f.T


def _format_table(embT, a0, na):
    ng = na // 2
    sd = jax.ShapeDtypeStruct((ng, _SBM, 128, 128), jnp.float32)
    return pl.pallas_call(
        _tr_block,
        grid=(ng, _TGRID),
        in_specs=[
            pl.BlockSpec((1, _D, _TBT), lambda g, c: (a0 + 2 * g, 0, c)),
            pl.BlockSpec((1, _D, _TBT), lambda g, c: (a0 + 2 * g + 1, 0, c)),
        ],
        out_specs=[
            pl.BlockSpec((1, 8, 128, 128), lambda g, c: (g, c, 0, 0)),
            pl.BlockSpec((1, 8, 128, 128), lambda g, c: (g, c, 0, 0)),
        ],
        out_shape=(sd, sd),
    )(embT, embT)


def _sc_gather(idxT, tev, tod, na, part_widths):
    mesh = plsc.VectorSubcoreMesh(core_axis_name="c", subcore_axis_name="s")

    @functools.partial(
        pl.kernel,
        mesh=mesh,
        compiler_params=pltpu.CompilerParams(use_tc_tiling_on_sc=False),
        out_type=tuple(jax.ShapeDtypeStruct((_B, w * _D), jnp.float32)
                       for w in part_widths),
        scratch_types=[
            pltpu.VMEM((na, _BW), jnp.int32),
            pltpu.VMEM((2, _BW, _D), jnp.float32),
            pltpu.SemaphoreType.DMA,
            pltpu.SemaphoreType.DMA,
        ],
    )
    def gather_kernel(idx_hbm, tev_hbm, tod_hbm, *rest):
        outs = rest[:len(part_widths)]
        idx_v, rows_v, sem0, sem1 = rest[len(part_widths):]
        wid = lax.axis_index("s") * _NC + lax.axis_index("c")
        b0 = wid * _BW
        pltpu.sync_copy(idx_hbm.at[:, pl.ds(b0, _BW)], idx_v)
        sems = (sem0, sem1)

        def issue(m):
            tab = tev_hbm if m % 2 == 0 else tod_hbm
            base = (m // 2) * _VPAD
            s = m % 2
            return [
                pltpu.async_copy(
                    tab.at[pl.ds(base, _VPAD)].at[
                        idx_v.at[m, pl.ds(j * _IDX_ROW, _IDX_ROW)]],
                    rows_v.at[s].at[pl.ds(j * _IDX_ROW, _IDX_ROW)],
                    sems[s],
                )
                for j in range(_SUB)
            ]

        gathers = issue(0)
        stores = [None, None]
        for m in range(na):
            s = m % 2
            nxt = None
            if m + 1 < na:
                if stores[1 - s] is not None:
                    stores[1 - s].wait()
                    stores[1 - s] = None
                nxt = issue(m + 1)
            for h in gathers:
                h.wait()
            gathers = nxt
            if stores[s] is not None:
                stores[s].wait()
            stores[s] = pltpu.async_copy(
                rows_v.at[s],
                outs[m // _PB].at[pl.ds(b0, _BW),
                                  pl.ds((m % _PB) * _D, _D)],
                sems[s],
            )
        for st in stores:
            if st is not None:
                st.wait()

    return gather_kernel(idxT, tev, tod)


_BB = 2048


def _mlp_block(x1_ref, x2_ref, x3_ref, x4_ref, w1_ref_, w2_ref_, w3_ref_,
               w4_ref_, b0_ref, w1_ref, b1_ref, w2_ref, b2_ref, wf_ref,
               bf_ref, o_ref):
    h = (jnp.dot(x1_ref[...], w1_ref_[...], preferred_element_type=jnp.float32)
         + jnp.dot(x2_ref[...], w2_ref_[...], preferred_element_type=jnp.float32)
         + jnp.dot(x3_ref[...], w3_ref_[...], preferred_element_type=jnp.float32)
         + jnp.dot(x4_ref[...], w4_ref_[...], preferred_element_type=jnp.float32)
         + b0_ref[...])
    h = jnp.maximum(h, 0.0)
    h = jnp.maximum(
        jnp.dot(h, w1_ref[...], preferred_element_type=jnp.float32) + b1_ref[...], 0.0) + h
    h = jnp.maximum(
        jnp.dot(h, w2_ref[...], preferred_element_type=jnp.float32) + b2_ref[...], 0.0) + h
    y = jnp.dot(h, wf_ref[...], preferred_element_type=jnp.float32) + bf_ref[...]
    o_ref[...] = -y[:, 0]


def _mlp(xs, w0s, b0, W1, b1, W2, b2, Wf, bf):
    full = lambda a: pl.BlockSpec(a.shape, lambda i: (0,) * a.ndim)
    return pl.pallas_call(
        _mlp_block,
        grid=(_B // _BB,),
        in_specs=(
            [pl.BlockSpec((_BB, x.shape[1]), lambda i: (i, 0)) for x in xs]
            + [full(w) for w in w0s]
            + [full(b0), full(W1), full(b1), full(W2), full(b2),
               full(Wf), full(bf)]
        ),
        out_specs=pl.BlockSpec((_BB,), lambda i: (i,)),
        out_shape=jax.ShapeDtypeStruct((_B,), jnp.float32),
    )(*xs, *w0s, b0, W1, b1, W2, b2, Wf, bf)


def kernel(inputs, emb, W0, b0, W1, b1, W2, b2, Wf, bf):
    embT = jnp.transpose(emb, (0, 2, 1))
    t = inputs.T
    rows = (t & -1024) + ((t & 127) << 3) + ((t >> 7) & 7)
    ta_ev, ta_od = _format_table(embT, 0, _MA)
    tb_ev, tb_od = _format_table(embT, _MA, _MB)
    x1, x2 = _sc_gather(rows[:_MA], ta_ev.reshape(-1, _D),
                        ta_od.reshape(-1, _D), _MA, (_PB, _PB))
    x3, x4 = _sc_gather(rows[_MA:], tb_ev.reshape(-1, _D),
                        tb_od.reshape(-1, _D), _MB, (_PB, _MB - _PB))
    w0s = (W0[0:128], W0[128:256], W0[256:384], W0[384:416])
    return _mlp((x1, x2, x3, x4), w0s, b0.reshape(1, _H), W1,
                b1.reshape(1, _H), W2, b2.reshape(1, _H), Wf,
                bf.reshape(1, 1))


# speedup vs baseline: 40.4805x; 1.1888x over previous
import functools

import jax
import jax.numpy as jnp
from jax import lax
from jax.experimental import pallas as pl
from jax.experimental.pallas import tpu as pltpu
from jax.experimental.pallas import tpu_sc as plsc

_B = 16384
_M = 26
_V = 100000
_D = 16
_H = 16

_NC = 2
_NS = 16
_NW = _NC * _NS
_BW = _B // _NW
_IDX_ROW = 128
_SUB = _BW // _IDX_ROW

_TSB = 1024
_TBT = 16384
_TGRID = -(-_V // _TBT)
_SBB = _TBT // _TSB
_SBM = _TGRID * _SBB
_VPAD = _SBM * _TSB

_MA = 16
_MB = _M - _MA
_PB = 8


def _tr_block(e0_ref, e1_ref, o0_ref, o1_ref):
    for e_ref, o_ref in ((e0_ref, o0_ref), (e1_ref, o1_ref)):
        for s in range(_SBB):
            ec = e_ref[0, :, s * _TSB:(s + 1) * _TSB]
            f = jnp.concatenate(
                [ec[:, a * 128:(a + 1) * 128] for a in range(8)], axis=0)
            o_ref[0, s, :, :] = f.T


def _format_table(embT, a0, na):
    ng = na // 2
    sd = jax.ShapeDtypeStruct((ng, _SBM, 128, 128), jnp.float32)
    return pl.pallas_call(
        _tr_block,
        grid=(ng, _TGRID),
        in_specs=[
            pl.BlockSpec((1, _D, _TBT), lambda g, c: (a0 + 2 * g, 0, c)),
            pl.BlockSpec((1, _D, _TBT), lambda g, c: (a0 + 2 * g + 1, 0, c)),
        ],
        out_specs=[
            pl.BlockSpec((1, _SBB, 128, 128), lambda g, c: (g, c, 0, 0)),
            pl.BlockSpec((1, _SBB, 128, 128), lambda g, c: (g, c, 0, 0)),
        ],
        out_shape=(sd, sd),
    )(embT, embT)


def _sc_gather(idxT, tev, tod, na, part_widths):
    mesh = plsc.VectorSubcoreMesh(core_axis_name="c", subcore_axis_name="s")

    @functools.partial(
        pl.kernel,
        mesh=mesh,
        compiler_params=pltpu.CompilerParams(use_tc_tiling_on_sc=False),
        out_type=tuple(jax.ShapeDtypeStruct((_B, w * _D), jnp.float32)
                       for w in part_widths),
        scratch_types=[
            pltpu.VMEM((na, _BW), jnp.int32),
            pltpu.VMEM((2, _BW, _D), jnp.float32),
            pltpu.SemaphoreType.DMA,
            pltpu.SemaphoreType.DMA,
        ],
    )
    def gather_kernel(idx_hbm, tev_hbm, tod_hbm, *rest):
        outs = rest[:len(part_widths)]
        idx_v, rows_v, sem0, sem1 = rest[len(part_widths):]
        wid = lax.axis_index("s") * _NC + lax.axis_index("c")
        b0 = wid * _BW
        pltpu.sync_copy(idx_hbm.at[:, pl.ds(b0, _BW)], idx_v)
        sems = (sem0, sem1)

        def issue(m):
            tab = tev_hbm if m % 2 == 0 else tod_hbm
            base = (m // 2) * _VPAD
            s = m % 2
            return [
                pltpu.async_copy(
                    tab.at[pl.ds(base, _VPAD)].at[
                        idx_v.at[m, pl.ds(j * _IDX_ROW, _IDX_ROW)]],
                    rows_v.at[s].at[pl.ds(j * _IDX_ROW, _IDX_ROW)],
                    sems[s],
                )
                for j in range(_SUB)
            ]

        gathers = issue(0)
        stores = [None, None]
        for m in range(na):
            s = m % 2
            nxt = None
            if m + 1 < na:
                if stores[1 - s] is not None:
                    stores[1 - s].wait()
                    stores[1 - s] = None
                nxt = issue(m + 1)
            for h in gathers:
                h.wait()
            gathers = nxt
            if stores[s] is not None:
                stores[s].wait()
            stores[s] = pltpu.async_copy(
                rows_v.at[s],
                outs[m // _PB].at[pl.ds(b0, _BW),
                                  pl.ds((m % _PB) * _D, _D)],
                sems[s],
            )
        for st in stores:
            if st is not None:
                st.wait()

    return gather_kernel(idxT, tev, tod)


_BB = 2048


def _mlp_block(x1_ref, x2_ref, x3_ref, x4_ref, w1_ref_, w2_ref_, w3_ref_,
               w4_ref_, b0_ref, w1_ref, b1_ref, w2_ref, b2_ref, wf_ref,
               bf_ref, o_ref):
    h = (jnp.dot(x1_ref[...], w1_ref_[...], preferred_element_type=jnp.float32)
         + jnp.dot(x2_ref[...], w2_ref_[...], preferred_element_type=jnp.float32)
         + jnp.dot(x3_ref[...], w3_ref_[...], preferred_element_type=jnp.float32)
         + jnp.dot(x4_ref[...], w4_ref_[...], preferred_element_type=jnp.float32)
         + b0_ref[...])
    h = jnp.maximum(h, 0.0)
    h = jnp.maximum(
        jnp.dot(h, w1_ref[...], preferred_element_type=jnp.float32) + b1_ref[...], 0.0) + h
    h = jnp.maximum(
        jnp.dot(h, w2_ref[...], preferred_element_type=jnp.float32) + b2_ref[...], 0.0) + h
    y = jnp.dot(h, wf_ref[...], preferred_element_type=jnp.float32) + bf_ref[...]
    o_ref[...] = -y[:, 0]


def _mlp(xs, w0s, b0, W1, b1, W2, b2, Wf, bf):
    full = lambda a: pl.BlockSpec(a.shape, lambda i: (0,) * a.ndim)
    return pl.pallas_call(
        _mlp_block,
        grid=(_B // _BB,),
        in_specs=(
            [pl.BlockSpec((_BB, x.shape[1]), lambda i: (i, 0)) for x in xs]
            + [full(w) for w in w0s]
            + [full(b0), full(W1), full(b1), full(W2), full(b2),
               full(Wf), full(bf)]
        ),
        out_specs=pl.BlockSpec((_BB,), lambda i: (i,)),
        out_shape=jax.ShapeDtypeStruct((_B,), jnp.float32),
    )(*xs, *w0s, b0, W1, b1, W2, b2, Wf, bf)


def kernel(inputs, emb, W0, b0, W1, b1, W2, b2, Wf, bf):
    embT = jnp.transpose(emb, (0, 2, 1))
    t = inputs.T
    rows = (t & -1024) + ((t & 127) << 3) + ((t >> 7) & 7)
    ta_ev, ta_od = _format_table(embT, 0, _MA)
    tb_ev, tb_od = _format_table(embT, _MA, _MB)
    x1, x2 = _sc_gather(rows[:_MA], ta_ev.reshape(-1, _D),
                        ta_od.reshape(-1, _D), _MA, (_PB, _PB))
    x3, x4 = _sc_gather(rows[_MA:], tb_ev.reshape(-1, _D),
                        tb_od.reshape(-1, _D), _MB, (_PB, _MB - _PB))
    w0s = (W0[0:128], W0[128:256], W0[256:384], W0[384:416])
    return _mlp((x1, x2, x3, x4), w0s, b0.reshape(1, _H), W1,
                b1.reshape(1, _H), W2, b2.reshape(1, _H), Wf,
                bf.reshape(1, 1))


# speedup vs baseline: 47.9360x; 1.1842x over previous
import functools

import jax
import jax.numpy as jnp
from jax import lax
from jax.experimental import pallas as pl
from jax.experimental.pallas import tpu as pltpu
from jax.experimental.pallas import tpu_sc as plsc

_B = 16384
_M = 26
_V = 100000
_D = 16
_H = 16

_NC = 2
_NS = 16
_NW = _NC * _NS
_BW = _B // _NW
_IDX_ROW = 128
_SUB = _BW // _IDX_ROW

_TSB = 1024
_TBT = 51200
_TGRID = -(-_V // _TBT)
_SBB = _TBT // _TSB
_SBM = _TGRID * _SBB
_VPAD = _SBM * _TSB

_MA = 16
_MB = _M - _MA
_PB = 8


def _tr_block(e0_ref, e1_ref, o0_ref, o1_ref):
    for e_ref, o_ref in ((e0_ref, o0_ref), (e1_ref, o1_ref)):
        for s in range(_SBB):
            ec = e_ref[0, :, s * _TSB:(s + 1) * _TSB]
            f = jnp.concatenate(
                [ec[:, a * 128:(a + 1) * 128] for a in range(8)], axis=0)
            o_ref[0, s, :, :] = f.T


def _format_table(embT, a0, na):
    ng = na // 2
    sd = jax.ShapeDtypeStruct((ng, _SBM, 128, 128), jnp.float32)
    return pl.pallas_call(
        _tr_block,
        grid=(ng, _TGRID),
        in_specs=[
            pl.BlockSpec((1, _D, _TBT), lambda g, c: (a0 + 2 * g, 0, c)),
            pl.BlockSpec((1, _D, _TBT), lambda g, c: (a0 + 2 * g + 1, 0, c)),
        ],
        out_specs=[
            pl.BlockSpec((1, _SBB, 128, 128), lambda g, c: (g, c, 0, 0)),
            pl.BlockSpec((1, _SBB, 128, 128), lambda g, c: (g, c, 0, 0)),
        ],
        out_shape=(sd, sd),
    )(embT, embT)


def _sc_gather(idxT, tev, tod, na, part_widths):
    mesh = plsc.VectorSubcoreMesh(core_axis_name="c", subcore_axis_name="s")

    @functools.partial(
        pl.kernel,
        mesh=mesh,
        compiler_params=pltpu.CompilerParams(use_tc_tiling_on_sc=False),
        out_type=tuple(jax.ShapeDtypeStruct((_B, w * _D), jnp.float32)
                       for w in part_widths),
        scratch_types=[
            pltpu.VMEM((na, _BW), jnp.int32),
            pltpu.VMEM((2, _BW, _D), jnp.float32),
            pltpu.SemaphoreType.DMA,
            pltpu.SemaphoreType.DMA,
        ],
    )
    def gather_kernel(idx_hbm, tev_hbm, tod_hbm, *rest):
        outs = rest[:len(part_widths)]
        idx_v, rows_v, sem0, sem1 = rest[len(part_widths):]
        wid = lax.axis_index("s") * _NC + lax.axis_index("c")
        b0 = wid * _BW
        pltpu.sync_copy(idx_hbm.at[:, pl.ds(b0, _BW)], idx_v)
        sems = (sem0, sem1)

        def issue(m):
            tab = tev_hbm if m % 2 == 0 else tod_hbm
            base = (m // 2) * _VPAD
            s = m % 2
            return [
                pltpu.async_copy(
                    tab.at[pl.ds(base, _VPAD)].at[
                        idx_v.at[m, pl.ds(j * _IDX_ROW, _IDX_ROW)]],
                    rows_v.at[s].at[pl.ds(j * _IDX_ROW, _IDX_ROW)],
                    sems[s],
                )
                for j in range(_SUB)
            ]

        gathers = issue(0)
        stores = [None, None]
        for m in range(na):
            s = m % 2
            nxt = None
            if m + 1 < na:
                if stores[1 - s] is not None:
                    stores[1 - s].wait()
                    stores[1 - s] = None
                nxt = issue(m + 1)
            for h in gathers:
                h.wait()
            gathers = nxt
            if stores[s] is not None:
                stores[s].wait()
            stores[s] = pltpu.async_copy(
                rows_v.at[s],
                outs[m // _PB].at[pl.ds(b0, _BW),
                                  pl.ds((m % _PB) * _D, _D)],
                sems[s],
            )
        for st in stores:
            if st is not None:
                st.wait()

    return gather_kernel(idxT, tev, tod)


_BB = 2048


def _mlp_block(x1_ref, x2_ref, x3_ref, x4_ref, w1_ref_, w2_ref_, w3_ref_,
               w4_ref_, b0_ref, w1_ref, b1_ref, w2_ref, b2_ref, wf_ref,
               bf_ref, o_ref):
    h = (jnp.dot(x1_ref[...], w1_ref_[...], preferred_element_type=jnp.float32)
         + jnp.dot(x2_ref[...], w2_ref_[...], preferred_element_type=jnp.float32)
         + jnp.dot(x3_ref[...], w3_ref_[...], preferred_element_type=jnp.float32)
         + jnp.dot(x4_ref[...], w4_ref_[...], preferred_element_type=jnp.float32)
         + b0_ref[...])
    h = jnp.maximum(h, 0.0)
    h = jnp.maximum(
        jnp.dot(h, w1_ref[...], preferred_element_type=jnp.float32) + b1_ref[...], 0.0) + h
    h = jnp.maximum(
        jnp.dot(h, w2_ref[...], preferred_element_type=jnp.float32) + b2_ref[...], 0.0) + h
    y = jnp.dot(h, wf_ref[...], preferred_element_type=jnp.float32) + bf_ref[...]
    o_ref[...] = -y[:, 0]


def _mlp(xs, w0s, b0, W1, b1, W2, b2, Wf, bf):
    full = lambda a: pl.BlockSpec(a.shape, lambda i: (0,) * a.ndim)
    return pl.pallas_call(
        _mlp_block,
        grid=(_B // _BB,),
        in_specs=(
            [pl.BlockSpec((_BB, x.shape[1]), lambda i: (i, 0)) for x in xs]
            + [full(w) for w in w0s]
            + [full(b0), full(W1), full(b1), full(W2), full(b2),
               full(Wf), full(bf)]
        ),
        out_specs=pl.BlockSpec((_BB,), lambda i: (i,)),
        out_shape=jax.ShapeDtypeStruct((_B,), jnp.float32),
    )(*xs, *w0s, b0, W1, b1, W2, b2, Wf, bf)


def kernel(inputs, emb, W0, b0, W1, b1, W2, b2, Wf, bf):
    embT = jnp.transpose(emb, (0, 2, 1))
    t = inputs.T
    rows = (t & -1024) + ((t & 127) << 3) + ((t >> 7) & 7)
    ta_ev, ta_od = _format_table(embT, 0, _MA)
    tb_ev, tb_od = _format_table(embT, _MA, _MB)
    x1, x2 = _sc_gather(rows[:_MA], ta_ev.reshape(-1, _D),
                        ta_od.reshape(-1, _D), _MA, (_PB, _PB))
    x3, x4 = _sc_gather(rows[_MA:], tb_ev.reshape(-1, _D),
                        tb_od.reshape(-1, _D), _MB, (_PB, _MB - _PB))
    w0s = (W0[0:128], W0[128:256], W0[256:384], W0[384:416])
    return _mlp((x1, x2, x3, x4), w0s, b0.reshape(1, _H), W1,
                b1.reshape(1, _H), W2, b2.reshape(1, _H), Wf,
                bf.reshape(1, 1))


# speedup vs baseline: 48.6475x; 1.0148x over previous
import functools

import jax
import jax.numpy as jnp
from jax import lax
from jax.experimental import pallas as pl
from jax.experimental.pallas import tpu as pltpu
from jax.experimental.pallas import tpu_sc as plsc

_B = 16384
_M = 26
_V = 100000
_D = 16
_H = 16

_NC = 2
_NS = 16
_NW = _NC * _NS
_BW = _B // _NW
_IDX_ROW = 128
_SUB = _BW // _IDX_ROW

_TSB = 1024
_TBT = 51200
_TGRID = -(-_V // _TBT)
_SBB = _TBT // _TSB
_SBM = _TGRID * _SBB
_VPAD = _SBM * _TSB

_MA = 16
_MB = 8
_MC = _M - _MA - _MB
_PB = 8


def _tr_block(e0_ref, e1_ref, o0_ref, o1_ref):
    for e_ref, o_ref in ((e0_ref, o0_ref), (e1_ref, o1_ref)):
        for s in range(_SBB):
            ec = e_ref[0, :, s * _TSB:(s + 1) * _TSB]
            f = jnp.concatenate(
                [ec[:, a * 128:(a + 1) * 128] for a in range(8)], axis=0)
            o_ref[0, s, :, :] = f.T


def _format_table(embT, a0, na):
    ng = na // 2
    sd = jax.ShapeDtypeStruct((ng, _SBM, 128, 128), jnp.float32)
    return pl.pallas_call(
        _tr_block,
        grid=(ng, _TGRID),
        in_specs=[
            pl.BlockSpec((1, _D, _TBT), lambda g, c: (a0 + 2 * g, 0, c)),
            pl.BlockSpec((1, _D, _TBT), lambda g, c: (a0 + 2 * g + 1, 0, c)),
        ],
        out_specs=[
            pl.BlockSpec((1, _SBB, 128, 128), lambda g, c: (g, c, 0, 0)),
            pl.BlockSpec((1, _SBB, 128, 128), lambda g, c: (g, c, 0, 0)),
        ],
        out_shape=(sd, sd),
    )(embT, embT)


def _sc_gather(idxT, tev, tod, na, part_widths):
    mesh = plsc.VectorSubcoreMesh(core_axis_name="c", subcore_axis_name="s")

    @functools.partial(
        pl.kernel,
        mesh=mesh,
        compiler_params=pltpu.CompilerParams(use_tc_tiling_on_sc=False),
        out_type=tuple(jax.ShapeDtypeStruct((_B, w * _D), jnp.float32)
                       for w in part_widths),
        scratch_types=[
            pltpu.VMEM((na, _BW), jnp.int32),
            pltpu.VMEM((2, _BW, _D), jnp.float32),
            pltpu.SemaphoreType.DMA,
            pltpu.SemaphoreType.DMA,
        ],
    )
    def gather_kernel(idx_hbm, tev_hbm, tod_hbm, *rest):
        outs = rest[:len(part_widths)]
        idx_v, rows_v, sem0, sem1 = rest[len(part_widths):]
        wid = lax.axis_index("s") * _NC + lax.axis_index("c")
        b0 = wid * _BW
        pltpu.sync_copy(idx_hbm.at[:, pl.ds(b0, _BW)], idx_v)
        sems = (sem0, sem1)

        def issue(m):
            tab = tev_hbm if m % 2 == 0 else tod_hbm
            base = (m // 2) * _VPAD
            s = m % 2
            return [
                pltpu.async_copy(
                    tab.at[pl.ds(base, _VPAD)].at[
                        idx_v.at[m, pl.ds(j * _IDX_ROW, _IDX_ROW)]],
                    rows_v.at[s].at[pl.ds(j * _IDX_ROW, _IDX_ROW)],
                    sems[s],
                )
                for j in range(_SUB)
            ]

        gathers = issue(0)
        stores = [None, None]
        for m in range(na):
            s = m % 2
            nxt = None
            if m + 1 < na:
                if stores[1 - s] is not None:
                    stores[1 - s].wait()
                    stores[1 - s] = None
                nxt = issue(m + 1)
            for h in gathers:
                h.wait()
            gathers = nxt
            if stores[s] is not None:
                stores[s].wait()
            stores[s] = pltpu.async_copy(
                rows_v.at[s],
                outs[m // _PB].at[pl.ds(b0, _BW),
                                  pl.ds((m % _PB) * _D, _D)],
                sems[s],
            )
        for st in stores:
            if st is not None:
                st.wait()

    return gather_kernel(idxT, tev, tod)


_BB = 2048


def _mlp_block(x1_ref, x2_ref, x3_ref, x4_ref, w1_ref_, w2_ref_, w3_ref_,
               w4_ref_, b0_ref, w1_ref, b1_ref, w2_ref, b2_ref, wf_ref,
               bf_ref, o_ref):
    h = (jnp.dot(x1_ref[...], w1_ref_[...], preferred_element_type=jnp.float32)
         + jnp.dot(x2_ref[...], w2_ref_[...], preferred_element_type=jnp.float32)
         + jnp.dot(x3_ref[...], w3_ref_[...], preferred_element_type=jnp.float32)
         + jnp.dot(x4_ref[...], w4_ref_[...], preferred_element_type=jnp.float32)
         + b0_ref[...])
    h = jnp.maximum(h, 0.0)
    h = jnp.maximum(
        jnp.dot(h, w1_ref[...], preferred_element_type=jnp.float32) + b1_ref[...], 0.0) + h
    h = jnp.maximum(
        jnp.dot(h, w2_ref[...], preferred_element_type=jnp.float32) + b2_ref[...], 0.0) + h
    y = jnp.dot(h, wf_ref[...], preferred_element_type=jnp.float32) + bf_ref[...]
    o_ref[...] = -y[:, 0]


def _mlp(xs, w0s, b0, W1, b1, W2, b2, Wf, bf):
    full = lambda a: pl.BlockSpec(a.shape, lambda i: (0,) * a.ndim)
    return pl.pallas_call(
        _mlp_block,
        grid=(_B // _BB,),
        in_specs=(
            [pl.BlockSpec((_BB, x.shape[1]), lambda i: (i, 0)) for x in xs]
            + [full(w) for w in w0s]
            + [full(b0), full(W1), full(b1), full(W2), full(b2),
               full(Wf), full(bf)]
        ),
        out_specs=pl.BlockSpec((_BB,), lambda i: (i,)),
        out_shape=jax.ShapeDtypeStruct((_B,), jnp.float32),
    )(*xs, *w0s, b0, W1, b1, W2, b2, Wf, bf)


def kernel(inputs, emb, W0, b0, W1, b1, W2, b2, Wf, bf):
    embT = jnp.transpose(emb, (0, 2, 1))
    t = inputs.T
    rows = (t & -1024) + ((t & 127) << 3) + ((t >> 7) & 7)
    ta_ev, ta_od = _format_table(embT, 0, _MA)
    tb_ev, tb_od = _format_table(embT, _MA, _MB)
    tc_ev, tc_od = _format_table(embT, _MA + _MB, _MC)
    x1, x2 = _sc_gather(rows[:_MA], ta_ev.reshape(-1, _D),
                        ta_od.reshape(-1, _D), _MA, (_PB, _PB))
    (x3,) = _sc_gather(rows[_MA:_MA + _MB], tb_ev.reshape(-1, _D),
                       tb_od.reshape(-1, _D), _MB, (_PB,))
    (x4,) = _sc_gather(rows[_MA + _MB:], tc_ev.reshape(-1, _D),
                       tc_od.reshape(-1, _D), _MC, (_MC,))
    w0s = (W0[0:128], W0[128:256], W0[256:384], W0[384:416])
    return _mlp((x1, x2, x3, x4), w0s, b0.reshape(1, _H), W1,
                b1.reshape(1, _H), W2, b2.reshape(1, _H), Wf,
                bf.reshape(1, 1))
